# baseline (speedup 1.0000x reference)
import functools

import jax
import jax.numpy as jnp
from jax import lax
from jax.experimental import pallas as pl
from jax.experimental.pallas import tpu as pltpu
from jax.experimental.pallas import tpu_sc as plsc

N = 10000
E = 160000
NPAD = 10240
EPAD = 163840
DIN = 256
DHID = 256
DOUT = 64
NC = 2
NS = 16
CHUNK = 128
RPT = NPAD // NS
BLK = 1024

_MESH = dict(core_axis_name="c", subcore_axis_name="s")


_DEG_CPT = EPAD // (NC * NS) // CHUNK


def _deg_body(dst_hbm, deg_out, didx, ones_v, zrow_v, acc, sem):
    c = lax.axis_index("c")
    s = lax.axis_index("s")

    def fill_ones(i, _):
        ones_v[pl.ds(i * 16, 16)] = jnp.full((16,), 1.0, jnp.float32)
        return 0

    lax.fori_loop(0, CHUNK // 16, fill_ones, 0)

    def fill_zero(i, _):
        zrow_v[pl.ds(i * 16, 16)] = jnp.zeros((16,), jnp.float32)
        return 0

    lax.fori_loop(0, RPT // 16, fill_zero, 0)
    pltpu.sync_copy(
        dst_hbm.at[pl.ds(c * (NC * _DEG_CPT * NS // 2) + s * _DEG_CPT, _DEG_CPT)],
        didx)
    pltpu.sync_copy(zrow_v, acc.at[pl.ds(s * RPT, RPT)])
    plsc.subcore_barrier()

    def issue(g, _):
        pltpu.async_copy(ones_v, acc.at[didx.at[g]], sem, add=True)
        return 0

    lax.fori_loop(0, _DEG_CPT, issue, 0)

    def drain(g, _):
        pltpu.make_async_copy(ones_v, acc.at[didx.at[0]], sem).wait()
        return 0

    lax.fori_loop(0, _DEG_CPT, drain, 0)
    plsc.subcore_barrier()
    pltpu.sync_copy(acc.at[pl.ds(s * RPT, RPT)],
                    deg_out.at[pl.ds(c * NPAD + s * RPT, RPT)])


_deg_kernel = pl.kernel(
    _deg_body,
    out_type=jax.ShapeDtypeStruct((NC * NPAD,), jnp.float32),
    mesh=plsc.VectorSubcoreMesh(**_MESH),
    scratch_types=[
        pltpu.MemorySpace.VMEM((_DEG_CPT, CHUNK), jnp.int32),
        pltpu.MemorySpace.VMEM((CHUNK,), jnp.float32),
        pltpu.MemorySpace.VMEM((RPT,), jnp.float32),
        pltpu.MemorySpace.VMEM_SHARED((NPAD,), jnp.float32),
        pltpu.SemaphoreType.DMA,
    ],
)


def _agg_body(src_cstride, dst_cstride, cpt, n_passes, nbuf,
              g_hbm, src_hbm, dst_hbm, z_hbm, out_hbm,
              sidx, didx, acc, *bufs):
    c = lax.axis_index("c")
    s = lax.axis_index("s")
    ppt = cpt // n_passes
    rows = bufs[:nbuf]
    sems = bufs[nbuf:]
    pltpu.sync_copy(z_hbm, acc.at[pl.ds(s * RPT, RPT)])
    plsc.subcore_barrier()

    for p in range(n_passes):
        sbase = c * src_cstride + s * cpt + p * ppt
        dbase = c * dst_cstride + s * cpt + p * ppt
        pltpu.sync_copy(src_hbm.at[pl.ds(sbase, ppt)], sidx)
        pltpu.sync_copy(dst_hbm.at[pl.ds(dbase, ppt)], didx)
        for b in range(nbuf):
            pltpu.async_copy(g_hbm.at[sidx.at[b]], rows[b], sems[b])

        def macro(m, _):
            for b in range(nbuf):
                ch = m * nbuf + b
                pltpu.make_async_copy(
                    g_hbm.at[sidx.at[ch]], rows[b], sems[b]).wait()
                pltpu.sync_copy(rows[b], acc.at[didx.at[ch]], add=True)
                pltpu.async_copy(g_hbm.at[sidx.at[ch + nbuf]], rows[b], sems[b])
            return 0

        lax.fori_loop(0, ppt // nbuf - 1, macro, 0)
        for b in range(nbuf):
            ch = ppt - nbuf + b
            pltpu.make_async_copy(g_hbm.at[sidx.at[ch]], rows[b], sems[b]).wait()
            pltpu.sync_copy(rows[b], acc.at[didx.at[ch]], add=True)

    plsc.subcore_barrier()
    pltpu.sync_copy(acc.at[pl.ds(s * RPT, RPT)],
                    out_hbm.at[pl.ds(c * NPAD + s * RPT, RPT)])


def _make_agg(width, src_cstride, dst_cstride, cpt, n_passes, nbuf):
    params = pltpu.CompilerParams(use_tc_tiling_on_sc=(width % 128 == 0))
    return pl.kernel(
        functools.partial(_agg_body, src_cstride, dst_cstride, cpt, n_passes,
                          nbuf),
        out_type=jax.ShapeDtypeStruct((NC * NPAD, width), jnp.float32),
        mesh=plsc.VectorSubcoreMesh(**_MESH),
        compiler_params=params,
        scratch_types=(
            [pltpu.MemorySpace.VMEM((cpt // n_passes, CHUNK), jnp.int32),
             pltpu.MemorySpace.VMEM((cpt // n_passes, CHUNK), jnp.int32),
             pltpu.MemorySpace.VMEM_SHARED((NPAD, width), jnp.float32)]
            + [pltpu.MemorySpace.VMEM((CHUNK, width), jnp.float32)
               for _ in range(nbuf)]
            + [pltpu.SemaphoreType.DMA for _ in range(nbuf)]
        ),
    )


_agg_l1 = _make_agg(128, src_cstride=EPAD // CHUNK, dst_cstride=0,
                    cpt=EPAD // NS // CHUNK, n_passes=2, nbuf=2)
_agg_l2 = _make_agg(DOUT, src_cstride=EPAD // NC // CHUNK,
                    dst_cstride=EPAD // NC // CHUNK,
                    cpt=EPAD // (NC * NS) // CHUNK, n_passes=1, nbuf=8)


def _dinv_block(deg_ref, i):
    d = deg_ref[0, pl.ds(i * BLK, BLK)] + deg_ref[1, pl.ds(i * BLK, BLK)] + 1.0
    return lax.rsqrt(d)


K2BLK = 2048


def _k2_body(x_ref, w_ref, deg_ref, out_ref):
    i = pl.program_id(0)
    d = (deg_ref[0, pl.ds(i * K2BLK, K2BLK)]
         + deg_ref[1, pl.ds(i * K2BLK, K2BLK)] + 1.0)
    dinv = lax.rsqrt(d)
    h = jnp.dot(x_ref[...], w_ref[...], preferred_element_type=jnp.float32)
    out_ref[...] = h * dinv[:, None]


def _k2_call(x, W1, deg):
    return pl.pallas_call(
        _k2_body,
        grid=(NPAD // K2BLK, 2),
        in_specs=[
            pl.BlockSpec((K2BLK, DIN), lambda i, j: (i, 0)),
            pl.BlockSpec((DIN, 128), lambda i, j: (0, j)),
            pl.BlockSpec((2, NPAD), lambda i, j: (0, 0)),
        ],
        out_specs=pl.BlockSpec((K2BLK, 128),
                               lambda i, j: (j * (NPAD // K2BLK) + i, 0)),
        out_shape=jax.ShapeDtypeStruct((NC * NPAD, 128), jnp.float32),
    )(x, W1, deg)


def _k4_body(alo, ahi, glo, ghi, mlo, mhi, deg_ref, b1_ref, w2_ref, out_ref):
    i = pl.program_id(0)
    dinv = _dinv_block(deg_ref, i)[:, None]
    zlo = jnp.where(mlo[0] != 0, 2.0 * jnp.maximum(
        dinv * (alo[0] + glo[0]) + b1_ref[0, pl.ds(0, 128)][None, :], 0.0), 0.0)
    zhi = jnp.where(mhi[0] != 0, 2.0 * jnp.maximum(
        dinv * (ahi[0] + ghi[0]) + b1_ref[0, pl.ds(128, 128)][None, :], 0.0), 0.0)
    h2 = (jnp.dot(zlo, w2_ref[pl.ds(0, 128), :], preferred_element_type=jnp.float32)
          + jnp.dot(zhi, w2_ref[pl.ds(128, 128), :], preferred_element_type=jnp.float32))
    out_ref[...] = h2 * dinv


def _k4_call(agg1, g1, mask3, deg, b1, W2):
    half = lambda h: pl.BlockSpec((1, BLK, 128), lambda i, h=h: (h, i, 0))
    return pl.pallas_call(
        _k4_body,
        grid=(NPAD // BLK,),
        in_specs=[
            half(0), half(1),
            half(0), half(1),
            half(0), half(1),
            pl.BlockSpec((2, NPAD), lambda i: (0, 0)),
            pl.BlockSpec((1, DHID), lambda i: (0, 0)),
            pl.BlockSpec((DHID, DOUT), lambda i: (0, 0)),
        ],
        out_specs=pl.BlockSpec((BLK, DOUT), lambda i: (i, 0)),
        out_shape=jax.ShapeDtypeStruct((NPAD, DOUT), jnp.float32),
    )(agg1, agg1, g1, g1, mask3, mask3, deg, b1, W2)


_K6_RPT = NPAD // (NC * NS)


def _k6sc_body(deg_hbm, p_hbm, g2_hbm, b2_hbm, out_hbm,
               d0, d1, dinv_v, p0, p1, g2v, b2v):
    c = lax.axis_index("c")
    s = lax.axis_index("s")
    w = s * NC + c
    r0 = w * _K6_RPT
    pltpu.sync_copy(deg_hbm.at[pl.ds(r0, _K6_RPT)], d0)
    pltpu.sync_copy(deg_hbm.at[pl.ds(NPAD + r0, _K6_RPT)], d1)
    pltpu.sync_copy(p_hbm.at[pl.ds(r0, _K6_RPT)], p0)
    pltpu.sync_copy(p_hbm.at[pl.ds(NPAD + r0, _K6_RPT)], p1)
    pltpu.sync_copy(g2_hbm.at[pl.ds(r0, _K6_RPT)], g2v)
    pltpu.sync_copy(b2_hbm, b2v)

    def mk_dinv(v, _):
        d = d0[pl.ds(v * 16, 16)] + d1[pl.ds(v * 16, 16)] + 1.0
        i = jax.lax.bitcast_convert_type(d, jnp.int32)
        i = jnp.int32(0x5F3759DF) - jax.lax.shift_right_logical(i, 1)
        y = jax.lax.bitcast_convert_type(i, jnp.float32)
        h = 0.5 * d
        for _ in range(3):
            y = y * (1.5 - h * y * y)
        dinv_v[pl.ds(v * 16, 16)] = y
        return 0

    lax.fori_loop(0, _K6_RPT // 16, mk_dinv, 0)

    b2vs = [b2v[pl.ds(k * 16, 16)] for k in range(DOUT // 16)]

    def row16(v, _):
        dv = dinv_v[pl.ds(v * 16, 16)]
        for j in range(16):
            r = v * 16 + j
            dr = dv[j]
            for k in range(DOUT // 16):
                sl = pl.ds(k * 16, 16)
                p0[r, sl] = (p0[r, sl] + p1[r, sl] + g2v[r, sl]) * dr + b2vs[k]
        return 0

    lax.fori_loop(0, _K6_RPT // 16, row16, 0)
    pltpu.sync_copy(p0, out_hbm.at[pl.ds(r0, _K6_RPT)])


_k6sc = pl.kernel(
    _k6sc_body,
    out_type=jax.ShapeDtypeStruct((NPAD, DOUT), jnp.float32),
    mesh=plsc.VectorSubcoreMesh(**_MESH),
    compiler_params=pltpu.CompilerParams(use_tc_tiling_on_sc=False),
    scratch_types=[
        pltpu.MemorySpace.VMEM((_K6_RPT,), jnp.float32),
        pltpu.MemorySpace.VMEM((_K6_RPT,), jnp.float32),
        pltpu.MemorySpace.VMEM((_K6_RPT,), jnp.float32),
        pltpu.MemorySpace.VMEM((_K6_RPT, DOUT), jnp.float32),
        pltpu.MemorySpace.VMEM((_K6_RPT, DOUT), jnp.float32),
        pltpu.MemorySpace.VMEM((_K6_RPT, DOUT), jnp.float32),
        pltpu.MemorySpace.VMEM((DOUT,), jnp.float32),
    ],
)


def kernel(x, edge_index, W1, b1, W2, b2):
    src = edge_index[0].astype(jnp.int32)
    dst = edge_index[1].astype(jnp.int32)
    pad_idx = (jnp.arange(EPAD - E, dtype=jnp.int32) % 192) + (NPAD - 192)
    srcp = jnp.concatenate([src, pad_idx])
    dstp = jnp.concatenate([dst, pad_idx])
    src2 = jnp.concatenate([srcp, srcp + NPAD])

    mask = jax.random.bernoulli(jax.random.key(42), 0.5, (N, DHID))
    mask_pad = jnp.pad(mask.astype(jnp.int8), ((0, NPAD - N), (0, 0)))
    mask3 = jnp.stack([mask_pad[:, :128], mask_pad[:, 128:]])
    z128 = jnp.zeros((RPT, 128), jnp.float32)
    z64 = jnp.zeros((RPT, DOUT), jnp.float32)

    dstp2 = dstp.reshape(-1, CHUNK)
    src22 = src2.reshape(-1, CHUNK)

    deg_flat = _deg_kernel(dstp2)
    deg = deg_flat.reshape(2, NPAD)
    g1 = _k2_call(x, W1, deg)
    agg1 = _agg_l1(g1, src22, dstp2, z128)
    g2 = _k4_call(agg1.reshape(2, NPAD, 128), g1.reshape(2, NPAD, 128),
                  mask3, deg, b1.reshape(1, DHID), W2)
    agg2 = _agg_l2(g2, src22, dstp2, z64)
    out = _k6sc(deg_flat, agg2, g2, b2)
    return out[:N]


# speedup vs baseline: 1.0011x; 1.0011x over previous
import functools

import jax
import jax.numpy as jnp
from jax import lax
from jax.experimental import pallas as pl
from jax.experimental.pallas import tpu as pltpu
from jax.experimental.pallas import tpu_sc as plsc

N = 10000
E = 160000
NPAD = 10240
EPAD = 163840
DIN = 256
DHID = 256
DOUT = 64
NC = 2
NS = 16
CHUNK = 128
RPT = NPAD // NS
BLK = 2048

_MESH = dict(core_axis_name="c", subcore_axis_name="s")


_DEG_CPT = EPAD // (NC * NS) // CHUNK


def _deg_body(dst_hbm, deg_out, didx, ones_v, zrow_v, acc, sem):
    c = lax.axis_index("c")
    s = lax.axis_index("s")

    def fill_ones(i, _):
        ones_v[pl.ds(i * 16, 16)] = jnp.full((16,), 1.0, jnp.float32)
        return 0

    lax.fori_loop(0, CHUNK // 16, fill_ones, 0)

    def fill_zero(i, _):
        zrow_v[pl.ds(i * 16, 16)] = jnp.zeros((16,), jnp.float32)
        return 0

    lax.fori_loop(0, RPT // 16, fill_zero, 0)
    pltpu.sync_copy(
        dst_hbm.at[pl.ds(c * (NC * _DEG_CPT * NS // 2) + s * _DEG_CPT, _DEG_CPT)],
        didx)
    pltpu.sync_copy(zrow_v, acc.at[pl.ds(s * RPT, RPT)])
    plsc.subcore_barrier()

    def issue(g, _):
        pltpu.async_copy(ones_v, acc.at[didx.at[g]], sem, add=True)
        return 0

    lax.fori_loop(0, _DEG_CPT, issue, 0)

    def drain(g, _):
        pltpu.make_async_copy(ones_v, acc.at[didx.at[0]], sem).wait()
        return 0

    lax.fori_loop(0, _DEG_CPT, drain, 0)
    plsc.subcore_barrier()
    pltpu.sync_copy(acc.at[pl.ds(s * RPT, RPT)],
                    deg_out.at[pl.ds(c * NPAD + s * RPT, RPT)])


_deg_kernel = pl.kernel(
    _deg_body,
    out_type=jax.ShapeDtypeStruct((NC * NPAD,), jnp.float32),
    mesh=plsc.VectorSubcoreMesh(**_MESH),
    scratch_types=[
        pltpu.MemorySpace.VMEM((_DEG_CPT, CHUNK), jnp.int32),
        pltpu.MemorySpace.VMEM((CHUNK,), jnp.float32),
        pltpu.MemorySpace.VMEM((RPT,), jnp.float32),
        pltpu.MemorySpace.VMEM_SHARED((NPAD,), jnp.float32),
        pltpu.SemaphoreType.DMA,
    ],
)


def _agg_body(src_cstride, dst_cstride, cpt, n_passes, nbuf,
              g_hbm, src_hbm, dst_hbm, z_hbm, out_hbm,
              sidx, didx, acc, *bufs):
    c = lax.axis_index("c")
    s = lax.axis_index("s")
    ppt = cpt // n_passes
    rows = bufs[:nbuf]
    sems = bufs[nbuf:]
    pltpu.sync_copy(z_hbm, acc.at[pl.ds(s * RPT, RPT)])
    plsc.subcore_barrier()

    for p in range(n_passes):
        sbase = c * src_cstride + s * cpt + p * ppt
        dbase = c * dst_cstride + s * cpt + p * ppt
        pltpu.sync_copy(src_hbm.at[pl.ds(sbase, ppt)], sidx)
        pltpu.sync_copy(dst_hbm.at[pl.ds(dbase, ppt)], didx)
        for b in range(nbuf):
            pltpu.async_copy(g_hbm.at[sidx.at[b]], rows[b], sems[b])

        def macro(m, _):
            for b in range(nbuf):
                ch = m * nbuf + b
                pltpu.make_async_copy(
                    g_hbm.at[sidx.at[ch]], rows[b], sems[b]).wait()
                pltpu.sync_copy(rows[b], acc.at[didx.at[ch]], add=True)
                pltpu.async_copy(g_hbm.at[sidx.at[ch + nbuf]], rows[b], sems[b])
            return 0

        lax.fori_loop(0, ppt // nbuf - 1, macro, 0)
        for b in range(nbuf):
            ch = ppt - nbuf + b
            pltpu.make_async_copy(g_hbm.at[sidx.at[ch]], rows[b], sems[b]).wait()
            pltpu.sync_copy(rows[b], acc.at[didx.at[ch]], add=True)

    plsc.subcore_barrier()
    pltpu.sync_copy(acc.at[pl.ds(s * RPT, RPT)],
                    out_hbm.at[pl.ds(c * NPAD + s * RPT, RPT)])


def _make_agg(width, src_cstride, dst_cstride, cpt, n_passes, nbuf):
    params = pltpu.CompilerParams(use_tc_tiling_on_sc=(width % 128 == 0))
    return pl.kernel(
        functools.partial(_agg_body, src_cstride, dst_cstride, cpt, n_passes,
                          nbuf),
        out_type=jax.ShapeDtypeStruct((NC * NPAD, width), jnp.float32),
        mesh=plsc.VectorSubcoreMesh(**_MESH),
        compiler_params=params,
        scratch_types=(
            [pltpu.MemorySpace.VMEM((cpt // n_passes, CHUNK), jnp.int32),
             pltpu.MemorySpace.VMEM((cpt // n_passes, CHUNK), jnp.int32),
             pltpu.MemorySpace.VMEM_SHARED((NPAD, width), jnp.float32)]
            + [pltpu.MemorySpace.VMEM((CHUNK, width), jnp.float32)
               for _ in range(nbuf)]
            + [pltpu.SemaphoreType.DMA for _ in range(nbuf)]
        ),
    )


_agg_l1 = _make_agg(128, src_cstride=EPAD // CHUNK, dst_cstride=0,
                    cpt=EPAD // NS // CHUNK, n_passes=2, nbuf=2)
_agg_l2 = _make_agg(DOUT, src_cstride=EPAD // NC // CHUNK,
                    dst_cstride=EPAD // NC // CHUNK,
                    cpt=EPAD // (NC * NS) // CHUNK, n_passes=1, nbuf=8)


def _dinv_block(deg_ref, i):
    d = deg_ref[0, pl.ds(i * BLK, BLK)] + deg_ref[1, pl.ds(i * BLK, BLK)] + 1.0
    return lax.rsqrt(d)


K2BLK = 2048


def _k2_body(x_ref, w_ref, deg_ref, out_ref):
    i = pl.program_id(0)
    d = (deg_ref[0, pl.ds(i * K2BLK, K2BLK)]
         + deg_ref[1, pl.ds(i * K2BLK, K2BLK)] + 1.0)
    dinv = lax.rsqrt(d)
    h = jnp.dot(x_ref[...], w_ref[...], preferred_element_type=jnp.float32)
    out_ref[...] = h * dinv[:, None]


def _k2_call(x, W1, deg):
    return pl.pallas_call(
        _k2_body,
        grid=(NPAD // K2BLK, 2),
        in_specs=[
            pl.BlockSpec((K2BLK, DIN), lambda i, j: (i, 0)),
            pl.BlockSpec((DIN, 128), lambda i, j: (0, j)),
            pl.BlockSpec((2, NPAD), lambda i, j: (0, 0)),
        ],
        out_specs=pl.BlockSpec((K2BLK, 128),
                               lambda i, j: (j * (NPAD // K2BLK) + i, 0)),
        out_shape=jax.ShapeDtypeStruct((NC * NPAD, 128), jnp.float32),
    )(x, W1, deg)


def _k4_body(alo, ahi, glo, ghi, mlo, mhi, deg_ref, b1_ref, w2_ref, out_ref):
    i = pl.program_id(0)
    dinv = _dinv_block(deg_ref, i)[:, None]
    zlo = jnp.where(mlo[0] != 0, 2.0 * jnp.maximum(
        dinv * (alo[0] + glo[0]) + b1_ref[0, pl.ds(0, 128)][None, :], 0.0), 0.0)
    zhi = jnp.where(mhi[0] != 0, 2.0 * jnp.maximum(
        dinv * (ahi[0] + ghi[0]) + b1_ref[0, pl.ds(128, 128)][None, :], 0.0), 0.0)
    h2 = (jnp.dot(zlo, w2_ref[pl.ds(0, 128), :], preferred_element_type=jnp.float32)
          + jnp.dot(zhi, w2_ref[pl.ds(128, 128), :], preferred_element_type=jnp.float32))
    out_ref[...] = h2 * dinv


def _k4_call(agg1, g1, mask3, deg, b1, W2):
    half = lambda h: pl.BlockSpec((1, BLK, 128), lambda i, h=h: (h, i, 0))
    return pl.pallas_call(
        _k4_body,
        grid=(NPAD // BLK,),
        in_specs=[
            half(0), half(1),
            half(0), half(1),
            half(0), half(1),
            pl.BlockSpec((2, NPAD), lambda i: (0, 0)),
            pl.BlockSpec((1, DHID), lambda i: (0, 0)),
            pl.BlockSpec((DHID, DOUT), lambda i: (0, 0)),
        ],
        out_specs=pl.BlockSpec((BLK, DOUT), lambda i: (i, 0)),
        out_shape=jax.ShapeDtypeStruct((NPAD, DOUT), jnp.float32),
    )(agg1, agg1, g1, g1, mask3, mask3, deg, b1, W2)


_K6_RPT = NPAD // (NC * NS)


def _k6sc_body(deg_hbm, p_hbm, g2_hbm, b2_hbm, out_hbm,
               d0, d1, dinv_v, p0, p1, g2v, b2v):
    c = lax.axis_index("c")
    s = lax.axis_index("s")
    w = s * NC + c
    r0 = w * _K6_RPT
    pltpu.sync_copy(deg_hbm.at[pl.ds(r0, _K6_RPT)], d0)
    pltpu.sync_copy(deg_hbm.at[pl.ds(NPAD + r0, _K6_RPT)], d1)
    pltpu.sync_copy(p_hbm.at[pl.ds(r0, _K6_RPT)], p0)
    pltpu.sync_copy(p_hbm.at[pl.ds(NPAD + r0, _K6_RPT)], p1)
    pltpu.sync_copy(g2_hbm.at[pl.ds(r0, _K6_RPT)], g2v)
    pltpu.sync_copy(b2_hbm, b2v)

    def mk_dinv(v, _):
        d = d0[pl.ds(v * 16, 16)] + d1[pl.ds(v * 16, 16)] + 1.0
        i = jax.lax.bitcast_convert_type(d, jnp.int32)
        i = jnp.int32(0x5F3759DF) - jax.lax.shift_right_logical(i, 1)
        y = jax.lax.bitcast_convert_type(i, jnp.float32)
        h = 0.5 * d
        for _ in range(3):
            y = y * (1.5 - h * y * y)
        dinv_v[pl.ds(v * 16, 16)] = y
        return 0

    lax.fori_loop(0, _K6_RPT // 16, mk_dinv, 0)

    b2vs = [b2v[pl.ds(k * 16, 16)] for k in range(DOUT // 16)]

    def row16(v, _):
        dv = dinv_v[pl.ds(v * 16, 16)]
        for j in range(16):
            r = v * 16 + j
            dr = dv[j]
            for k in range(DOUT // 16):
                sl = pl.ds(k * 16, 16)
                p0[r, sl] = (p0[r, sl] + p1[r, sl] + g2v[r, sl]) * dr + b2vs[k]
        return 0

    lax.fori_loop(0, _K6_RPT // 16, row16, 0)
    pltpu.sync_copy(p0, out_hbm.at[pl.ds(r0, _K6_RPT)])


_k6sc = pl.kernel(
    _k6sc_body,
    out_type=jax.ShapeDtypeStruct((NPAD, DOUT), jnp.float32),
    mesh=plsc.VectorSubcoreMesh(**_MESH),
    compiler_params=pltpu.CompilerParams(use_tc_tiling_on_sc=False),
    scratch_types=[
        pltpu.MemorySpace.VMEM((_K6_RPT,), jnp.float32),
        pltpu.MemorySpace.VMEM((_K6_RPT,), jnp.float32),
        pltpu.MemorySpace.VMEM((_K6_RPT,), jnp.float32),
        pltpu.MemorySpace.VMEM((_K6_RPT, DOUT), jnp.float32),
        pltpu.MemorySpace.VMEM((_K6_RPT, DOUT), jnp.float32),
        pltpu.MemorySpace.VMEM((_K6_RPT, DOUT), jnp.float32),
        pltpu.MemorySpace.VMEM((DOUT,), jnp.float32),
    ],
)


def kernel(x, edge_index, W1, b1, W2, b2):
    src = edge_index[0].astype(jnp.int32)
    dst = edge_index[1].astype(jnp.int32)
    pad_idx = (jnp.arange(EPAD - E, dtype=jnp.int32) % 192) + (NPAD - 192)
    srcp = jnp.concatenate([src, pad_idx])
    dstp = jnp.concatenate([dst, pad_idx])
    src2 = jnp.concatenate([srcp, srcp + NPAD])

    mask = jax.random.bernoulli(jax.random.key(42), 0.5, (N, DHID))
    mask_pad = jnp.pad(mask.astype(jnp.int8), ((0, NPAD - N), (0, 0)))
    mask3 = jnp.stack([mask_pad[:, :128], mask_pad[:, 128:]])
    z128 = jnp.zeros((RPT, 128), jnp.float32)
    z64 = jnp.zeros((RPT, DOUT), jnp.float32)

    dstp2 = dstp.reshape(-1, CHUNK)
    src22 = src2.reshape(-1, CHUNK)

    deg_flat = _deg_kernel(dstp2)
    deg = deg_flat.reshape(2, NPAD)
    g1 = _k2_call(x, W1, deg)
    agg1 = _agg_l1(g1, src22, dstp2, z128)
    g2 = _k4_call(agg1.reshape(2, NPAD, 128), g1.reshape(2, NPAD, 128),
                  mask3, deg, b1.reshape(1, DHID), W2)
    agg2 = _agg_l2(g2, src22, dstp2, z64)
    out = _k6sc(deg_flat, agg2, g2, b2)
    return out[:N]
